# baseline (speedup 1.0000x reference)
import functools

import jax
import jax.numpy as jnp
from jax import lax
from jax.experimental import pallas as pl
from jax.experimental.pallas import tpu as pltpu
from jax.experimental.pallas import tpu_sc as plsc

D = 128
B = 1024
T = 200
ROWS = B * T
NC = 2
NS = 16
L = 16
NW = NC * NS
RPW = ROWS // NW
CH = 128
NCH = RPW // CH
PDUP = 328
NB = 4


def _body(idx_hbm, tok_hbm, pos_hbm, out_hbm, idx_v, pos_v, buf, semg, sems):
  wid = lax.axis_index("s") * NC + lax.axis_index("c")
  pltpu.sync_copy(idx_hbm.at[wid], idx_v)
  pltpu.sync_copy(pos_hbm, pos_v)

  def gather_args(j):
    b = j % NB
    return (tok_hbm.at[idx_v.at[j]], buf.at[b], semg.at[b])

  def scatter_args(j):
    b = j % NB
    return (buf.at[b], out_hbm.at[pl.ds(wid * RPW + j * CH, CH)], sems.at[b])

  pltpu.async_copy(*gather_args(0))
  pltpu.async_copy(*gather_args(1))

  def chunk_body(j, carry):
    b = j % NB
    @pl.when(j + 2 < NCH)
    def _refill():
      @pl.when(j >= 2)
      def _drain():
        pltpu.make_async_copy(*scatter_args(j - 2)).wait()

      pltpu.async_copy(*gather_args(j + 2))

    pltpu.make_async_copy(*gather_args(j)).wait()

    poff = lax.rem(j * CH, T)
    RU = 2

    def add_row(r0, c2):
      for u in range(RU):
        r = r0 * RU + u
        vals = [pos_v[poff + r, pl.ds(c * L, L)] for c in range(D // L)]
        for c in range(D // L):
          plsc.addupdate(buf.at[b, r, pl.ds(c * L, L)], vals[c])
      return c2

    lax.fori_loop(0, CH // RU, add_row, 0)
    pltpu.async_copy(*scatter_args(j))
    return carry

  lax.fori_loop(0, NCH, chunk_body, 0)
  for j in range(NCH - 4, NCH):
    pltpu.make_async_copy(*scatter_args(j)).wait()


_mesh = plsc.VectorSubcoreMesh(core_axis_name="c", subcore_axis_name="s")

_call = functools.partial(
    pl.kernel,
    mesh=_mesh,
    out_type=jax.ShapeDtypeStruct((ROWS, D), jnp.float32),
    scratch_types=[
        pltpu.VMEM((NCH, CH), jnp.int32),
        pltpu.VMEM((PDUP, D), jnp.float32),
        pltpu.VMEM((NB, CH, D), jnp.float32),
        pltpu.SemaphoreType.DMA((NB,)),
        pltpu.SemaphoreType.DMA((NB,)),
    ],
)(_body)


@jax.jit
def kernel(idx, token_table, pos_table):
  idx3 = idx.reshape(NW, NCH, CH).astype(jnp.int32)
  pos2 = jnp.concatenate([pos_table[:T], pos_table[: PDUP - T]], axis=0)
  out = _call(idx3, token_table, pos2)
  return out.reshape(B, T, D)
